# Initial kernel scaffold; baseline (speedup 1.0000x reference)
import jax
import jax.numpy as jnp
from jax.experimental import pallas as pl


def kernel(node_attr, edge_index, edge_attr, W0, b0, gamma, beta, Wf, bf, Ws, bs):
    raise NotImplementedError("write your pallas kernel here")


# speedup vs baseline: 1.5733x; 1.5733x over previous
import functools

import jax
import jax.numpy as jnp
from jax import lax
from jax.experimental import pallas as pl
from jax.experimental.pallas import tpu as pltpu
from jax.experimental.pallas import tpu_sc as plsc

N = 10000
E = 160000
D = 256
ED = 16
Z = 2 * D + ED

RB = 1000
EB = 1000

NC = 2
NS = 16
NW = NC * NS

GK = 128
GKG = 112
G_PER = E // NW
G_FULL = G_PER // GKG
G_LAST = (G_PER - GKG) // 8 * 8

S_PER = E // NS
S_FULL = S_PER // GK
S_TAIL = S_PER - S_FULL * GK

DH = D // 2
TROW = 624


def _mm_stats_body(x_ref, w_ref, b_ref, y_ref, st_ref):
    i = pl.program_id(0)
    y = jnp.dot(x_ref[...], w_ref[...], preferred_element_type=jnp.float32)
    y = y + b_ref[...]
    y_ref[...] = y
    s1 = jnp.sum(y, axis=0)
    s2 = jnp.sum(y * y, axis=0)
    rows = lax.broadcasted_iota(jnp.int32, (8, D), 0)
    upd = jnp.where(rows == 0, s1[None, :], 0.0) + jnp.where(rows == 1, s2[None, :], 0.0)

    @pl.when(i == 0)
    def _():
        st_ref[...] = jnp.zeros_like(st_ref)

    st_ref[...] += upd


def _mm_stats(node_attr, W0, b0):
    return pl.pallas_call(
        _mm_stats_body,
        grid=(N // RB,),
        in_specs=[
            pl.BlockSpec((RB, D), lambda i: (i, 0)),
            pl.BlockSpec((D, D), lambda i: (0, 0)),
            pl.BlockSpec((1, D), lambda i: (0, 0)),
        ],
        out_specs=[
            pl.BlockSpec((RB, D), lambda i: (i, 0)),
            pl.BlockSpec((8, D), lambda i: (0, 0)),
        ],
        out_shape=[
            jax.ShapeDtypeStruct((N, D), jnp.float32),
            jax.ShapeDtypeStruct((8, D), jnp.float32),
        ],
    )(node_attr, W0, b0.reshape(1, D))


def _bn_relu_body(y_ref, st_ref, g_ref, b_ref, x_ref):
    mean = st_ref[0:1, :] / N
    var = st_ref[1:2, :] / N - mean * mean
    inv = g_ref[...] * lax.rsqrt(var + 1e-5)
    x_ref[...] = jnp.maximum((y_ref[...] - mean) * inv + b_ref[...], 0.0)


def _bn_relu(y, st, gamma, beta):
    return pl.pallas_call(
        _bn_relu_body,
        grid=(N // RB,),
        in_specs=[
            pl.BlockSpec((RB, D), lambda i: (i, 0)),
            pl.BlockSpec((8, D), lambda i: (0, 0)),
            pl.BlockSpec((1, D), lambda i: (0, 0)),
            pl.BlockSpec((1, D), lambda i: (0, 0)),
        ],
        out_specs=pl.BlockSpec((RB, D), lambda i: (i, 0)),
        out_shape=jax.ShapeDtypeStruct((N, D), jnp.float32),
    )(y, st, gamma.reshape(1, D), beta.reshape(1, D))


def _project_body(x_ref, wt_ref, wu_ref, bt_ref, t_ref, u_ref):
    x = x_ref[...]
    t_ref[...] = jnp.dot(x, wt_ref[...], preferred_element_type=jnp.float32) + bt_ref[...]
    u_ref[...] = jnp.dot(x, wu_ref[...], preferred_element_type=jnp.float32)


def _project(x, W_T, W_U, b_T):
    return pl.pallas_call(
        _project_body,
        grid=(N // RB,),
        in_specs=[
            pl.BlockSpec((RB, D), lambda i: (i, 0)),
            pl.BlockSpec((D, 2 * D), lambda i: (0, 0)),
            pl.BlockSpec((D, 2 * D), lambda i: (0, 0)),
            pl.BlockSpec((1, 2 * D), lambda i: (0, 0)),
        ],
        out_specs=[
            pl.BlockSpec((RB, 2 * D), lambda i: (i, 0)),
            pl.BlockSpec((RB, 2 * D), lambda i: (i, 0)),
        ],
        out_shape=[
            jax.ShapeDtypeStruct((N, 2 * D), jnp.float32),
            jax.ShapeDtypeStruct((N, 2 * D), jnp.float32),
        ],
    )(x, W_T, W_U, b_T)


def _edge_body(g_ref, ea_ref, we_ref, m_ref):
    ep = jnp.dot(ea_ref[...], we_ref[...], preferred_element_type=jnp.float32)
    gz = g_ref[...] + ep
    f = gz[:, :D]
    s = gz[:, D:]
    sig = 1.0 / (1.0 + jnp.exp(-f))
    sp = jnp.maximum(s, 0.0) + jnp.log(1.0 + jnp.exp(-jnp.abs(s)))
    msg = sig * sp
    m_ref[...] = jnp.stack([msg[:, :DH], msg[:, DH:]], axis=0)


def _edge_stage(G, edge_attr, W_E):
    return pl.pallas_call(
        _edge_body,
        grid=(E // EB,),
        in_specs=[
            pl.BlockSpec((EB, 2 * D), lambda i: (i, 0)),
            pl.BlockSpec((EB, ED), lambda i: (i, 0)),
            pl.BlockSpec((ED, 2 * D), lambda i: (0, 0)),
        ],
        out_specs=pl.BlockSpec((2, EB, DH), lambda i: (0, i, 0)),
        out_shape=jax.ShapeDtypeStruct((2, E, DH), jnp.float32),
    )(G, edge_attr, W_E)


def _update_body(a_ref, x_ref, o_ref):
    a = jnp.concatenate([a_ref[0], a_ref[1]], axis=1)
    o_ref[...] = jnp.maximum(a + x_ref[...], 0.0)


def _update(aggr2, x):
    return pl.pallas_call(
        _update_body,
        grid=(N // RB,),
        in_specs=[
            pl.BlockSpec((2, RB, DH), lambda i: (0, i, 0)),
            pl.BlockSpec((RB, D), lambda i: (i, 0)),
        ],
        out_specs=pl.BlockSpec((RB, D), lambda i: (i, 0)),
        out_shape=jax.ShapeDtypeStruct((N, D), jnp.float32),
    )(aggr2, x)


def _gather_body(t_hbm, u_hbm, dst_hbm, src_hbm, g_hbm,
                 dbuf, sbuf, gbuf, gbuf2, sem, sem2):
    wid = lax.axis_index("s") * NC + lax.axis_index("c")
    base = wid * G_PER

    def chunk(off):
        pltpu.sync_copy(dst_hbm.at[pl.ds(off, GKG)], dbuf)
        pltpu.sync_copy(src_hbm.at[pl.ds(off, GKG)], sbuf)
        ct = pltpu.async_copy(t_hbm.at[dbuf], gbuf, sem)
        cu = pltpu.async_copy(u_hbm.at[sbuf], gbuf2, sem2)
        ct.wait()
        cu.wait()

        def addrow(r, _):
            for k in range(2 * D // 16):
                sl = pl.ds(k * 16, 16)
                gbuf[r, sl] = gbuf[r, sl] + gbuf2[r, sl]
            return 0

        lax.fori_loop(0, GKG, addrow, 0)
        pltpu.sync_copy(gbuf, g_hbm.at[pl.ds(off, GKG)])

    def body(j, _):
        chunk(base + j * GKG)
        return 0

    lax.fori_loop(0, G_FULL, body, 0)
    chunk(base + G_LAST)


def _gather(T, U, dst, src):
    mesh = plsc.VectorSubcoreMesh(core_axis_name="c", subcore_axis_name="s")
    k = pl.kernel(
        _gather_body,
        out_type=jax.ShapeDtypeStruct((E, 2 * D), jnp.float32),
        mesh=mesh,
        scratch_types=[
            pltpu.VMEM((GKG,), jnp.int32),
            pltpu.VMEM((GKG,), jnp.int32),
            pltpu.VMEM((GKG, 2 * D), jnp.float32),
            pltpu.VMEM((GKG, 2 * D), jnp.float32),
            pltpu.SemaphoreType.DMA,
            pltpu.SemaphoreType.DMA,
        ],
    )
    return k(T, U, dst, src)


def _scatter_body(m_hbm, dst_hbm, a_hbm, spbuf, mbuf, ibuf, mbuf_t, ibuf_t):
    c = lax.axis_index("c")
    sid = lax.axis_index("s")

    def zrow(i, _):
        r = i // (DH // 16)
        k = i % (DH // 16)
        mbuf[r, pl.ds(k * 16, 16)] = jnp.zeros((16,), jnp.float32)
        return 0

    lax.fori_loop(0, GK * (DH // 16), zrow, 0)

    start = sid * TROW
    for t in range(TROW // GK):
        pltpu.sync_copy(mbuf, spbuf.at[pl.ds(start + t * GK, GK)])
    pltpu.sync_copy(mbuf.at[pl.ds(0, TROW - (TROW // GK) * GK)],
                    spbuf.at[pl.ds(start + (TROW // GK) * GK,
                                   TROW - (TROW // GK) * GK)])

    @pl.when(sid == NS - 1)
    def _():
        pltpu.sync_copy(mbuf.at[pl.ds(0, N - NS * TROW)],
                        spbuf.at[pl.ds(NS * TROW, N - NS * TROW)])

    plsc.subcore_barrier()

    base = sid * S_PER

    def chunk(j, _):
        off = base + j * GK
        pltpu.sync_copy(dst_hbm.at[pl.ds(off, GK)], ibuf)
        pltpu.sync_copy(m_hbm.at[c, pl.ds(off, GK)], mbuf)
        pltpu.sync_copy(mbuf, spbuf.at[ibuf], add=True)
        return 0

    lax.fori_loop(0, S_FULL, chunk, 0)

    off = base + S_FULL * GK
    pltpu.sync_copy(dst_hbm.at[pl.ds(off, S_TAIL)], ibuf_t)
    pltpu.sync_copy(m_hbm.at[c, pl.ds(off, S_TAIL)], mbuf_t)
    pltpu.sync_copy(mbuf_t, spbuf.at[ibuf_t], add=True)

    plsc.subcore_barrier()
    pltpu.sync_copy(spbuf.at[pl.ds(sid * TROW, TROW)],
                    a_hbm.at[c, pl.ds(sid * TROW, TROW)])

    @pl.when(sid == NS - 1)
    def _():
        pltpu.sync_copy(spbuf.at[pl.ds(NS * TROW, N - NS * TROW)],
                        a_hbm.at[c, pl.ds(NS * TROW, N - NS * TROW)])


def _scatter(msg2, dst):
    mesh = plsc.VectorSubcoreMesh(core_axis_name="c", subcore_axis_name="s")
    k = pl.kernel(
        _scatter_body,
        out_type=jax.ShapeDtypeStruct((2, N, DH), jnp.float32),
        mesh=mesh,
        scratch_types=[
            pltpu.VMEM_SHARED((N, DH), jnp.float32),
            pltpu.VMEM((GK, DH), jnp.float32),
            pltpu.VMEM((GK,), jnp.int32),
            pltpu.VMEM((S_TAIL, DH), jnp.float32),
            pltpu.VMEM((S_TAIL,), jnp.int32),
        ],
    )
    return k(msg2, dst)


def kernel(node_attr, edge_index, edge_attr, W0, b0, gamma, beta, Wf, bf, Ws, bs):
    src = edge_index[0]
    dst = edge_index[1]

    y, st = _mm_stats(node_attr, W0, b0)
    x = _bn_relu(y, st, gamma, beta)

    L = Wf.shape[0]
    for l in range(L):
        W_T = jnp.concatenate([Wf[l, :D], Ws[l, :D]], axis=1)
        W_U = jnp.concatenate([Wf[l, D:2 * D], Ws[l, D:2 * D]], axis=1)
        W_E = jnp.concatenate([Wf[l, 2 * D:], Ws[l, 2 * D:]], axis=1)
        b_T = jnp.concatenate([bf[l], bs[l]]).reshape(1, 2 * D)
        T, U = _project(x, W_T, W_U, b_T)
        G = _gather(T, U, dst, src)
        msg2 = _edge_stage(G, edge_attr, W_E)
        aggr2 = _scatter(msg2, dst)
        x = _update(aggr2, x)
    return x
